# baseline (speedup 1.0000x reference)
import functools

import jax
import jax.numpy as jnp
from jax import lax
from jax.experimental import pallas as pl
from jax.experimental.pallas import tpu as pltpu
from jax.experimental.pallas import tpu_sc as plsc

NC = 2
NS = 16
NW = NC * NS
CHUNK = 128


def _seg_sum_body(h_hbm, src_hbm, dst_hbm, zblk_hbm, out_hbm,
                  src_v, dst_v, rows_a, rows_b, acc_sh, sem_a, sem_b):
    n_half = src_hbm.shape[1] // 2
    cpt2 = src_v.shape[0]
    n_pad = acc_sh.shape[0]
    rows_per_tile = n_pad // NS
    cid = lax.axis_index("c")
    sid = lax.axis_index("s")
    wid = cid * NS + sid

    row0 = sid * rows_per_tile
    for r in range(rows_per_tile // CHUNK):
        pltpu.sync_copy(zblk_hbm, acc_sh.at[pl.ds(row0 + r * CHUNK, CHUNK)])
    plsc.subcore_barrier()

    for half in range(2):
        pltpu.sync_copy(src_hbm.at[wid, pl.ds(half * n_half, cpt2)], src_v)
        pltpu.sync_copy(dst_hbm.at[wid, pl.ds(half * n_half, cpt2)], dst_v)

        def pair_body(j2, carry):
            j = 2 * j2
            g0 = pltpu.async_copy(h_hbm.at[src_v.at[j]], rows_a, sem_a)
            g1 = pltpu.async_copy(h_hbm.at[src_v.at[j + 1]], rows_b, sem_b)
            g0.wait()
            pltpu.sync_copy(rows_a, acc_sh.at[dst_v.at[j]], add=True)
            g1.wait()
            pltpu.sync_copy(rows_b, acc_sh.at[dst_v.at[j + 1]], add=True)
            return carry

        lax.fori_loop(0, cpt2 // 2, pair_body, 0)
    plsc.subcore_barrier()

    pltpu.sync_copy(acc_sh.at[pl.ds(row0, rows_per_tile)],
                    out_hbm.at[cid, pl.ds(row0, rows_per_tile)])


def _segment_sum_sc(h, src, dst, n_pad):
    n, d = h.shape
    e = src.shape[0]
    cpt = -(-e // (NW * CHUNK))
    cpt = -(-cpt // 4) * 4
    ept = cpt * CHUNK
    e_pad = ept * NW
    src_p = jnp.concatenate(
        [src, jnp.zeros((e_pad - e,), jnp.int32)]).reshape(NW, cpt, CHUNK)
    dst_p = jnp.concatenate(
        [dst, jnp.full((e_pad - e,), n, jnp.int32)]).reshape(NW, cpt, CHUNK)
    zblk = jnp.zeros((CHUNK, d), jnp.float32)

    mesh = plsc.VectorSubcoreMesh(core_axis_name="c", subcore_axis_name="s",
                                  num_cores=NC, num_subcores=NS)
    grid_kernel = functools.partial(
        pl.kernel,
        out_type=jax.ShapeDtypeStruct((NC, n_pad, d), jnp.float32),
        mesh=mesh,
        scratch_types=[
            pltpu.VMEM((cpt // 2, CHUNK), jnp.int32),
            pltpu.VMEM((cpt // 2, CHUNK), jnp.int32),
            pltpu.VMEM((CHUNK, d), jnp.float32),
            pltpu.VMEM((CHUNK, d), jnp.float32),
            pltpu.VMEM_SHARED((n_pad, d), jnp.float32),
            pltpu.SemaphoreType.DMA,
            pltpu.SemaphoreType.DMA,
        ],
    )
    return grid_kernel(_seg_sum_body)(h, src_p, dst_p, zblk)


def _gin_tc_body(h_ref, parts_ref, w1_ref, b1_ref, w2_ref, b2_ref,
                 eps_ref, g_ref, bt_ref, out_ref):
    n = h_ref.shape[0]
    h = h_ref[...]
    neigh = parts_ref[0, :n, :] + parts_ref[1, :n, :]
    z = (1.0 + eps_ref[0, 0]) * h + neigh
    z = jnp.dot(z, w1_ref[...], preferred_element_type=jnp.float32)
    z = jnp.maximum(z + b1_ref[...], 0.0)
    z = jnp.dot(z, w2_ref[...], preferred_element_type=jnp.float32)
    z = z + b2_ref[...]
    mean = jnp.mean(z, axis=0, keepdims=True)
    var = jnp.mean((z - mean) ** 2, axis=0, keepdims=True)
    z = (z - mean) * lax.rsqrt(var + 1e-5) * g_ref[...] + bt_ref[...]
    out_ref[...] = h + jnp.maximum(z, 0.0)


def kernel(h, edge_index, W1, b1, W2, b2, eps, bn_gamma, bn_beta):
    n, d = h.shape
    n_pad = -(-(n + 1) // (NS * CHUNK)) * NS * CHUNK
    parts = _segment_sum_sc(h, edge_index[0], edge_index[1], n_pad)
    return pl.pallas_call(
        _gin_tc_body,
        out_shape=jax.ShapeDtypeStruct((n, d), jnp.float32),
    )(h, parts,
      W1, b1.reshape(1, d), W2, b2.reshape(1, d),
      jnp.reshape(eps, (1, 1)), bn_gamma.reshape(1, d), bn_beta.reshape(1, d))


# speedup vs baseline: 1.0484x; 1.0484x over previous
import functools

import jax
import jax.numpy as jnp
from jax import lax
from jax.experimental import pallas as pl
from jax.experimental.pallas import tpu as pltpu
from jax.experimental.pallas import tpu_sc as plsc

NC = 2
NS = 16
CHUNK = 128
CORE0_SHARE = 2.0 / 3.0


def _seg_sum_body(cpt0, cpt1, h_hbm, src_hbm, dst_hbm, zblk_hbm, out_hbm,
                  src_v, dst_v, rows_v, acc_sh, sem):
    n_pad = acc_sh.shape[0]
    rows_per_tile = n_pad // NS
    cid = lax.axis_index("c")
    sid = lax.axis_index("s")

    row0 = sid * rows_per_tile
    for r in range(rows_per_tile // CHUNK):
        pltpu.sync_copy(zblk_hbm, acc_sh.at[pl.ds(row0 + r * CHUNK, CHUNK)])

    cpt_c = jnp.where(cid == 0, cpt0, cpt1)
    base = pl.multiple_of(cid * NS * cpt0 + sid * cpt_c, 8)
    pltpu.sync_copy(src_hbm.at[pl.ds(base, cpt0)], src_v)
    pltpu.sync_copy(dst_hbm.at[pl.ds(base, cpt0)], dst_v)
    plsc.subcore_barrier()

    def chunk_body(j, carry):
        pltpu.async_copy(h_hbm.at[src_v.at[j]], rows_v, sem).wait()
        pltpu.sync_copy(rows_v, acc_sh.at[dst_v.at[j]], add=True)
        return carry

    lax.fori_loop(0, cpt_c, chunk_body, 0)
    plsc.subcore_barrier()

    pltpu.sync_copy(acc_sh.at[pl.ds(row0, rows_per_tile)],
                    out_hbm.at[cid, pl.ds(row0, rows_per_tile)])


def _segment_sum_sc(h, src, dst, n_pad):
    n, d = h.shape
    e = src.shape[0]
    tc_real = -(-e // CHUNK)
    cpt0 = max(8, round(tc_real * CORE0_SHARE / (NS * 8)) * 8)
    cpt1 = max(8, -(-(tc_real - NS * cpt0) // (NS * 8)) * 8)
    covered = NS * (cpt0 + cpt1)
    ch_alloc = covered + max(0, cpt0 - cpt1)
    e_alloc = ch_alloc * CHUNK
    src_p = jnp.concatenate(
        [src, jnp.zeros((e_alloc - e,), jnp.int32)]).reshape(ch_alloc, CHUNK)
    dst_p = jnp.concatenate(
        [dst, jnp.full((e_alloc - e,), n, jnp.int32)]).reshape(ch_alloc, CHUNK)
    zblk = jnp.zeros((CHUNK, d), jnp.float32)

    mesh = plsc.VectorSubcoreMesh(core_axis_name="c", subcore_axis_name="s",
                                  num_cores=NC, num_subcores=NS)
    grid_kernel = functools.partial(
        pl.kernel,
        out_type=jax.ShapeDtypeStruct((NC, n_pad, d), jnp.float32),
        mesh=mesh,
        scratch_types=[
            pltpu.VMEM((cpt0, CHUNK), jnp.int32),
            pltpu.VMEM((cpt0, CHUNK), jnp.int32),
            pltpu.VMEM((CHUNK, d), jnp.float32),
            pltpu.VMEM_SHARED((n_pad, d), jnp.float32),
            pltpu.SemaphoreType.DMA,
        ],
    )
    body = functools.partial(_seg_sum_body, cpt0, cpt1)
    return grid_kernel(body)(h, src_p, dst_p, zblk)


def _gin_tc_body(h_ref, parts_ref, w1_ref, b1_ref, w2_ref, b2_ref,
                 eps_ref, g_ref, bt_ref, out_ref):
    n = h_ref.shape[0]
    h = h_ref[...]
    neigh = parts_ref[0, :n, :] + parts_ref[1, :n, :]
    z = (1.0 + eps_ref[0, 0]) * h + neigh
    z = jnp.dot(z, w1_ref[...], preferred_element_type=jnp.float32)
    z = jnp.maximum(z + b1_ref[...], 0.0)
    z = jnp.dot(z, w2_ref[...], preferred_element_type=jnp.float32)
    z = z + b2_ref[...]
    mean = jnp.mean(z, axis=0, keepdims=True)
    var = jnp.mean((z - mean) ** 2, axis=0, keepdims=True)
    z = (z - mean) * lax.rsqrt(var + 1e-5) * g_ref[...] + bt_ref[...]
    out_ref[...] = h + jnp.maximum(z, 0.0)


def kernel(h, edge_index, W1, b1, W2, b2, eps, bn_gamma, bn_beta):
    n, d = h.shape
    n_pad = -(-(n + 1) // (NS * CHUNK)) * NS * CHUNK
    parts = _segment_sum_sc(h, edge_index[0], edge_index[1], n_pad)
    return pl.pallas_call(
        _gin_tc_body,
        out_shape=jax.ShapeDtypeStruct((n, d), jnp.float32),
    )(h, parts,
      W1, b1.reshape(1, d), W2, b2.reshape(1, d),
      jnp.reshape(eps, (1, 1)), bn_gamma.reshape(1, d), bn_beta.reshape(1, d))


# speedup vs baseline: 2.8071x; 2.6776x over previous
import functools

import jax
import jax.numpy as jnp
from jax import lax
from jax.experimental import pallas as pl
from jax.experimental.pallas import tpu as pltpu
from jax.experimental.pallas import tpu_sc as plsc

NC = 2
NS = 16
NW = NC * NS
CHUNK = 128


def _seg_sum_body(h_hbm, src_hbm, dst_hbm, zblk_hbm, out_hbm,
                  src_v, dst_v, rows_v, acc_sh, sem):
    cpt = src_v.shape[0]
    n_pad = acc_sh.shape[0]
    rows_per_tile = n_pad // NS
    cid = lax.axis_index("c")
    sid = lax.axis_index("s")
    wid = cid * NS + sid

    pltpu.sync_copy(src_hbm.at[wid], src_v)
    pltpu.sync_copy(dst_hbm.at[wid], dst_v)

    row0 = sid * rows_per_tile
    for r in range(rows_per_tile // CHUNK):
        pltpu.sync_copy(zblk_hbm, acc_sh.at[pl.ds(row0 + r * CHUNK, CHUNK)])
    plsc.subcore_barrier()

    def chunk_body(j, carry):
        pltpu.async_copy(h_hbm.at[src_v.at[j]], rows_v, sem).wait()
        pltpu.sync_copy(rows_v, acc_sh.at[dst_v.at[j]], add=True)
        return carry

    lax.fori_loop(0, cpt, chunk_body, 0)
    plsc.subcore_barrier()

    pltpu.sync_copy(acc_sh.at[pl.ds(row0, rows_per_tile)],
                    out_hbm.at[cid, pl.ds(row0, rows_per_tile)])


def _segment_sum_sc(h, src, dst, n_pad):
    n, d = h.shape
    e = src.shape[0]
    ept = -(-e // (NW * CHUNK)) * CHUNK
    e_pad = ept * NW
    cpt = ept // CHUNK
    n_fill = e_pad - e
    fill = jnp.arange(n_fill, dtype=jnp.int32)
    src_p = jnp.concatenate([src, fill % CHUNK]).reshape(NW, cpt, CHUNK)
    dst_p = jnp.concatenate(
        [dst, n + fill % (n_pad - n)]).reshape(NW, cpt, CHUNK)
    zblk = jnp.zeros((CHUNK, d), jnp.float32)

    mesh = plsc.VectorSubcoreMesh(core_axis_name="c", subcore_axis_name="s",
                                  num_cores=NC, num_subcores=NS)
    grid_kernel = functools.partial(
        pl.kernel,
        out_type=jax.ShapeDtypeStruct((NC, n_pad, d), jnp.float32),
        mesh=mesh,
        scratch_types=[
            pltpu.VMEM((cpt, CHUNK), jnp.int32),
            pltpu.VMEM((cpt, CHUNK), jnp.int32),
            pltpu.VMEM((CHUNK, d), jnp.float32),
            pltpu.VMEM_SHARED((n_pad, d), jnp.float32),
            pltpu.SemaphoreType.DMA,
        ],
    )
    return grid_kernel(_seg_sum_body)(h, src_p, dst_p, zblk)


def _gin_tc_body(h_ref, parts_ref, w1_ref, b1_ref, w2_ref, b2_ref,
                 eps_ref, g_ref, bt_ref, out_ref):
    n = h_ref.shape[0]
    h = h_ref[...]
    neigh = parts_ref[0, :n, :] + parts_ref[1, :n, :]
    z = (1.0 + eps_ref[0, 0]) * h + neigh
    z = jnp.dot(z, w1_ref[...], preferred_element_type=jnp.float32)
    z = jnp.maximum(z + b1_ref[...], 0.0)
    z = jnp.dot(z, w2_ref[...], preferred_element_type=jnp.float32)
    z = z + b2_ref[...]
    mean = jnp.mean(z, axis=0, keepdims=True)
    var = jnp.mean((z - mean) ** 2, axis=0, keepdims=True)
    z = (z - mean) * lax.rsqrt(var + 1e-5) * g_ref[...] + bt_ref[...]
    out_ref[...] = h + jnp.maximum(z, 0.0)


def kernel(h, edge_index, W1, b1, W2, b2, eps, bn_gamma, bn_beta):
    n, d = h.shape
    n_pad = -(-(n + 1) // (NS * CHUNK)) * NS * CHUNK
    parts = _segment_sum_sc(h, edge_index[0], edge_index[1], n_pad)
    return pl.pallas_call(
        _gin_tc_body,
        out_shape=jax.ShapeDtypeStruct((n, d), jnp.float32),
    )(h, parts,
      W1, b1.reshape(1, d), W2, b2.reshape(1, d),
      jnp.reshape(eps, (1, 1)), bn_gamma.reshape(1, d), bn_beta.reshape(1, d))


# speedup vs baseline: 3.1317x; 1.1157x over previous
import functools

import jax
import jax.numpy as jnp
from jax import lax
from jax.experimental import pallas as pl
from jax.experimental.pallas import tpu as pltpu
from jax.experimental.pallas import tpu_sc as plsc

NC = 2
NS = 16
NW = NC * NS
CHUNK = 128


def _seg_sum_body(h_hbm, src_hbm, dst_hbm, zblk_hbm, out_hbm,
                  src_v, dst_v, rows_a, rows_b, acc_sh, sem_a, sem_b):
    n_half = src_hbm.shape[1] // 2
    cpt2 = src_v.shape[0]
    n_pad = acc_sh.shape[0]
    rows_per_tile = n_pad // NS
    cid = lax.axis_index("c")
    sid = lax.axis_index("s")
    wid = cid * NS + sid

    row0 = sid * rows_per_tile
    for r in range(rows_per_tile // CHUNK):
        pltpu.sync_copy(zblk_hbm, acc_sh.at[pl.ds(row0 + r * CHUNK, CHUNK)])
    plsc.subcore_barrier()

    for half in range(2):
        pltpu.sync_copy(src_hbm.at[wid, pl.ds(half * n_half, cpt2)], src_v)
        pltpu.sync_copy(dst_hbm.at[wid, pl.ds(half * n_half, cpt2)], dst_v)

        def pair_body(j2, carry):
            j = 2 * j2
            g0 = pltpu.async_copy(h_hbm.at[src_v.at[j]], rows_a, sem_a)
            g1 = pltpu.async_copy(h_hbm.at[src_v.at[j + 1]], rows_b, sem_b)
            g0.wait()
            pltpu.sync_copy(rows_a, acc_sh.at[dst_v.at[j]], add=True)
            g1.wait()
            pltpu.sync_copy(rows_b, acc_sh.at[dst_v.at[j + 1]], add=True)
            return carry

        lax.fori_loop(0, cpt2 // 2, pair_body, 0)
    plsc.subcore_barrier()

    pltpu.sync_copy(acc_sh.at[pl.ds(row0, rows_per_tile)],
                    out_hbm.at[cid, pl.ds(row0, rows_per_tile)])


def _segment_sum_sc(h, src, dst, n_pad):
    n, d = h.shape
    e = src.shape[0]
    cpt = -(-e // (NW * CHUNK))
    cpt = -(-cpt // 4) * 4
    ept = cpt * CHUNK
    e_pad = ept * NW
    n_fill = e_pad - e
    fill = jnp.arange(n_fill, dtype=jnp.int32)
    src_p = jnp.concatenate([src, fill % CHUNK]).reshape(NW, cpt, CHUNK)
    dst_p = jnp.concatenate(
        [dst, n + fill % (n_pad - n)]).reshape(NW, cpt, CHUNK)
    zblk = jnp.zeros((CHUNK, d), jnp.float32)

    mesh = plsc.VectorSubcoreMesh(core_axis_name="c", subcore_axis_name="s",
                                  num_cores=NC, num_subcores=NS)
    grid_kernel = functools.partial(
        pl.kernel,
        out_type=jax.ShapeDtypeStruct((NC, n_pad, d), jnp.float32),
        mesh=mesh,
        scratch_types=[
            pltpu.VMEM((cpt // 2, CHUNK), jnp.int32),
            pltpu.VMEM((cpt // 2, CHUNK), jnp.int32),
            pltpu.VMEM((CHUNK, d), jnp.float32),
            pltpu.VMEM((CHUNK, d), jnp.float32),
            pltpu.VMEM_SHARED((n_pad, d), jnp.float32),
            pltpu.SemaphoreType.DMA,
            pltpu.SemaphoreType.DMA,
        ],
    )
    return grid_kernel(_seg_sum_body)(h, src_p, dst_p, zblk)


def _gin_tc_body(h_ref, parts_ref, w1_ref, b1_ref, w2_ref, b2_ref,
                 eps_ref, g_ref, bt_ref, out_ref):
    n = h_ref.shape[0]
    h = h_ref[...]
    neigh = parts_ref[0, :n, :] + parts_ref[1, :n, :]
    z = (1.0 + eps_ref[0, 0]) * h + neigh
    z = jnp.dot(z, w1_ref[...], preferred_element_type=jnp.float32)
    z = jnp.maximum(z + b1_ref[...], 0.0)
    z = jnp.dot(z, w2_ref[...], preferred_element_type=jnp.float32)
    z = z + b2_ref[...]
    mean = jnp.mean(z, axis=0, keepdims=True)
    var = jnp.mean((z - mean) ** 2, axis=0, keepdims=True)
    z = (z - mean) * lax.rsqrt(var + 1e-5) * g_ref[...] + bt_ref[...]
    out_ref[...] = h + jnp.maximum(z, 0.0)


def kernel(h, edge_index, W1, b1, W2, b2, eps, bn_gamma, bn_beta):
    n, d = h.shape
    n_pad = -(-(n + 1) // (NS * CHUNK)) * NS * CHUNK
    parts = _segment_sum_sc(h, edge_index[0], edge_index[1], n_pad)
    return pl.pallas_call(
        _gin_tc_body,
        out_shape=jax.ShapeDtypeStruct((n, d), jnp.float32),
    )(h, parts,
      W1, b1.reshape(1, d), W2, b2.reshape(1, d),
      jnp.reshape(eps, (1, 1)), bn_gamma.reshape(1, d), bn_beta.reshape(1, d))


# speedup vs baseline: 3.9330x; 1.2559x over previous
import functools

import jax
import jax.numpy as jnp
from jax import lax
from jax.experimental import pallas as pl
from jax.experimental.pallas import tpu as pltpu
from jax.experimental.pallas import tpu_sc as plsc

NC = 2
NS = 16
NW = NC * NS
CHUNK = 128


def _seg_sum_body(h_hbm, src_hbm, dst_hbm, zblk_hbm, out_hbm,
                  src_v, dst_v, rows_a, rows_b, acc_sh, sem_a, sem_b):
    n_half = src_hbm.shape[1] // 2
    cpt2 = src_v.shape[0]
    n_pad = acc_sh.shape[0]
    rows_per_tile = n_pad // NS
    cid = lax.axis_index("c")
    sid = lax.axis_index("s")
    wid = cid * NS + sid

    row0 = sid * rows_per_tile
    for r in range(rows_per_tile // CHUNK):
        pltpu.sync_copy(zblk_hbm, acc_sh.at[pl.ds(row0 + r * CHUNK, CHUNK)])
    plsc.subcore_barrier()

    bufs = (rows_a, rows_b)
    sems = (sem_a, sem_b)
    for half in range(2):
        pltpu.sync_copy(src_hbm.at[wid, pl.ds(half * n_half, cpt2)], src_v)
        pltpu.sync_copy(dst_hbm.at[wid, pl.ds(half * n_half, cpt2)], dst_v)
        pltpu.async_copy(h_hbm.at[src_v.at[0]], rows_a, sem_a)
        pltpu.async_copy(h_hbm.at[src_v.at[1]], rows_b, sem_b)

        def pair_body(j2, carry):
            for b in range(2):
                j = 2 * j2 + b
                pltpu.make_async_copy(
                    h_hbm.at[src_v.at[j]], bufs[b], sems[b]).wait()
                pltpu.sync_copy(bufs[b], acc_sh.at[dst_v.at[j]], add=True)
                pltpu.async_copy(h_hbm.at[src_v.at[j + 2]], bufs[b], sems[b])
            return carry

        lax.fori_loop(0, cpt2 // 2 - 1, pair_body, 0)
        for b in range(2):
            j = cpt2 - 2 + b
            pltpu.make_async_copy(
                h_hbm.at[src_v.at[j]], bufs[b], sems[b]).wait()
            pltpu.sync_copy(bufs[b], acc_sh.at[dst_v.at[j]], add=True)
    plsc.subcore_barrier()

    pltpu.sync_copy(acc_sh.at[pl.ds(row0, rows_per_tile)],
                    out_hbm.at[cid, pl.ds(row0, rows_per_tile)])


def _segment_sum_sc(h, src, dst, n_pad):
    n, d = h.shape
    e = src.shape[0]
    cpt = -(-e // (NW * CHUNK))
    cpt = -(-cpt // 4) * 4
    ept = cpt * CHUNK
    e_pad = ept * NW
    n_fill = e_pad - e
    fill = jnp.arange(n_fill, dtype=jnp.int32)
    src_p = jnp.concatenate([src, fill % CHUNK]).reshape(NW, cpt, CHUNK)
    dst_p = jnp.concatenate(
        [dst, n + fill % (n_pad - n)]).reshape(NW, cpt, CHUNK)
    zblk = jnp.zeros((CHUNK, d), jnp.float32)

    mesh = plsc.VectorSubcoreMesh(core_axis_name="c", subcore_axis_name="s",
                                  num_cores=NC, num_subcores=NS)
    grid_kernel = functools.partial(
        pl.kernel,
        out_type=jax.ShapeDtypeStruct((NC, n_pad, d), jnp.float32),
        mesh=mesh,
        scratch_types=[
            pltpu.VMEM((cpt // 2, CHUNK), jnp.int32),
            pltpu.VMEM((cpt // 2, CHUNK), jnp.int32),
            pltpu.VMEM((CHUNK, d), jnp.float32),
            pltpu.VMEM((CHUNK, d), jnp.float32),
            pltpu.VMEM_SHARED((n_pad, d), jnp.float32),
            pltpu.SemaphoreType.DMA,
            pltpu.SemaphoreType.DMA,
        ],
    )
    return grid_kernel(_seg_sum_body)(h, src_p, dst_p, zblk)


def _gin_tc_body(h_ref, parts_ref, w1_ref, b1_ref, w2_ref, b2_ref,
                 eps_ref, g_ref, bt_ref, out_ref):
    n = h_ref.shape[0]
    h = h_ref[...]
    neigh = parts_ref[0, :n, :] + parts_ref[1, :n, :]
    z = (1.0 + eps_ref[0, 0]) * h + neigh
    z = jnp.dot(z, w1_ref[...], preferred_element_type=jnp.float32)
    z = jnp.maximum(z + b1_ref[...], 0.0)
    z = jnp.dot(z, w2_ref[...], preferred_element_type=jnp.float32)
    z = z + b2_ref[...]
    mean = jnp.mean(z, axis=0, keepdims=True)
    var = jnp.mean((z - mean) ** 2, axis=0, keepdims=True)
    z = (z - mean) * lax.rsqrt(var + 1e-5) * g_ref[...] + bt_ref[...]
    out_ref[...] = h + jnp.maximum(z, 0.0)


def kernel(h, edge_index, W1, b1, W2, b2, eps, bn_gamma, bn_beta):
    n, d = h.shape
    n_pad = -(-(n + 1) // (NS * CHUNK)) * NS * CHUNK
    parts = _segment_sum_sc(h, edge_index[0], edge_index[1], n_pad)
    return pl.pallas_call(
        _gin_tc_body,
        out_shape=jax.ShapeDtypeStruct((n, d), jnp.float32),
    )(h, parts,
      W1, b1.reshape(1, d), W2, b2.reshape(1, d),
      jnp.reshape(eps, (1, 1)), bn_gamma.reshape(1, d), bn_beta.reshape(1, d))


# speedup vs baseline: 3.9675x; 1.0088x over previous
import functools

import jax
import jax.numpy as jnp
from jax import lax
from jax.experimental import pallas as pl
from jax.experimental.pallas import tpu as pltpu
from jax.experimental.pallas import tpu_sc as plsc

NC = 2
NS = 16
NW = NC * NS
CHUNK = 128


def _seg_sum_body(last_tile, cpt,
                  h_hbm, src_hbm, dst_hbm, tsrc_hbm, tdst_hbm, zblk_hbm,
                  out_hbm, src_v, dst_v, rows_a, rows_b, acc_sh, sem_a, sem_b):
    cpt2 = src_v.shape[0]
    n_pad = acc_sh.shape[0]
    rows_per_tile = n_pad // NS
    cid = lax.axis_index("c")
    sid = lax.axis_index("s")
    wid = cid * NS + sid

    row0 = sid * rows_per_tile
    for r in range(rows_per_tile // CHUNK):
        pltpu.sync_copy(zblk_hbm, acc_sh.at[pl.ds(row0 + r * CHUNK, CHUNK)])
    plsc.subcore_barrier()

    bufs = (rows_a, rows_b)
    sems = (sem_a, sem_b)
    for half in range(2):
        chunk0 = pl.multiple_of(wid * cpt + half * cpt2, 8)

        @pl.when(wid < last_tile)
        def _stage_real():
            pltpu.sync_copy(src_hbm.at[pl.ds(chunk0, cpt2)], src_v)
            pltpu.sync_copy(dst_hbm.at[pl.ds(chunk0, cpt2)], dst_v)

        @pl.when(wid >= last_tile)
        def _stage_tail():
            toff = pl.multiple_of(chunk0 - last_tile * cpt, 8)
            pltpu.sync_copy(tsrc_hbm.at[pl.ds(toff, cpt2)], src_v)
            pltpu.sync_copy(tdst_hbm.at[pl.ds(toff, cpt2)], dst_v)
        pltpu.async_copy(h_hbm.at[src_v.at[0]], rows_a, sem_a)
        pltpu.async_copy(h_hbm.at[src_v.at[1]], rows_b, sem_b)

        def pair_body(j2, carry):
            for b in range(2):
                j = 2 * j2 + b
                pltpu.make_async_copy(
                    h_hbm.at[src_v.at[j]], bufs[b], sems[b]).wait()
                pltpu.sync_copy(bufs[b], acc_sh.at[dst_v.at[j]], add=True)
                pltpu.async_copy(h_hbm.at[src_v.at[j + 2]], bufs[b], sems[b])
            return carry

        lax.fori_loop(0, cpt2 // 2 - 1, pair_body, 0)
        for b in range(2):
            j = cpt2 - 2 + b
            pltpu.make_async_copy(
                h_hbm.at[src_v.at[j]], bufs[b], sems[b]).wait()
            pltpu.sync_copy(bufs[b], acc_sh.at[dst_v.at[j]], add=True)
    plsc.subcore_barrier()

    pltpu.sync_copy(acc_sh.at[pl.ds(row0, rows_per_tile)],
                    out_hbm.at[cid, pl.ds(row0, rows_per_tile)])


def _segment_sum_sc(h, src, dst, n_pad):
    n, d = h.shape
    e = src.shape[0]
    cpt = -(-e // (NW * CHUNK))
    cpt = -(-cpt // 4) * 4
    ept = cpt * CHUNK
    e_pad = ept * NW
    last_tile = e // ept
    full_e = (e // CHUNK) * CHUNK
    src_full = src[:full_e].reshape(-1, CHUNK)
    dst_full = dst[:full_e].reshape(-1, CHUNK)
    if last_tile < NW:
        n_fill = e_pad - e
        fill = jnp.arange(n_fill, dtype=jnp.int32)
        tail_src = jnp.concatenate(
            [src[last_tile * ept:], fill % CHUNK]).reshape(-1, CHUNK)
        tail_dst = jnp.concatenate(
            [dst[last_tile * ept:], n + fill % (n_pad - n)]).reshape(-1, CHUNK)
    else:
        tail_src = jnp.zeros((cpt, CHUNK), jnp.int32)
        tail_dst = jnp.full((cpt, CHUNK), n, jnp.int32)
    zblk = jnp.zeros((CHUNK, d), jnp.float32)

    mesh = plsc.VectorSubcoreMesh(core_axis_name="c", subcore_axis_name="s",
                                  num_cores=NC, num_subcores=NS)
    grid_kernel = functools.partial(
        pl.kernel,
        out_type=jax.ShapeDtypeStruct((NC, n_pad, d), jnp.float32),
        mesh=mesh,
        scratch_types=[
            pltpu.VMEM((cpt // 2, CHUNK), jnp.int32),
            pltpu.VMEM((cpt // 2, CHUNK), jnp.int32),
            pltpu.VMEM((CHUNK, d), jnp.float32),
            pltpu.VMEM((CHUNK, d), jnp.float32),
            pltpu.VMEM_SHARED((n_pad, d), jnp.float32),
            pltpu.SemaphoreType.DMA,
            pltpu.SemaphoreType.DMA,
        ],
    )
    body = functools.partial(_seg_sum_body, last_tile, cpt)
    return grid_kernel(body)(h, src_full, dst_full, tail_src, tail_dst, zblk)


def _gin_tc_body(h_ref, parts_ref, w1_ref, b1_ref, w2_ref, b2_ref,
                 eps_ref, g_ref, bt_ref, out_ref):
    n = h_ref.shape[0]
    h = h_ref[...]
    neigh = parts_ref[0, :n, :] + parts_ref[1, :n, :]
    z = (1.0 + eps_ref[0, 0]) * h + neigh
    z = jnp.dot(z, w1_ref[...], preferred_element_type=jnp.float32)
    z = jnp.maximum(z + b1_ref[...], 0.0)
    z = jnp.dot(z, w2_ref[...], preferred_element_type=jnp.float32)
    z = z + b2_ref[...]
    mean = jnp.mean(z, axis=0, keepdims=True)
    var = jnp.mean((z - mean) ** 2, axis=0, keepdims=True)
    z = (z - mean) * lax.rsqrt(var + 1e-5) * g_ref[...] + bt_ref[...]
    out_ref[...] = h + jnp.maximum(z, 0.0)


def kernel(h, edge_index, W1, b1, W2, b2, eps, bn_gamma, bn_beta):
    n, d = h.shape
    n_pad = -(-(n + 1) // (NS * CHUNK)) * NS * CHUNK
    parts = _segment_sum_sc(h, edge_index[0], edge_index[1], n_pad)
    return pl.pallas_call(
        _gin_tc_body,
        out_shape=jax.ShapeDtypeStruct((n, d), jnp.float32),
    )(h, parts,
      W1, b1.reshape(1, d), W2, b2.reshape(1, d),
      jnp.reshape(eps, (1, 1)), bn_gamma.reshape(1, d), bn_beta.reshape(1, d))


# speedup vs baseline: 4.1457x; 1.0449x over previous
import functools

import jax
import jax.numpy as jnp
from jax import lax
from jax.experimental import pallas as pl
from jax.experimental.pallas import tpu as pltpu
from jax.experimental.pallas import tpu_sc as plsc

NC = 2
NS = 16
NW = NC * NS
CHUNK = 128


def _seg_sum_body(last_tile, cpt,
                  h_hbm, ei_hbm, tsrc_hbm, tdst_hbm, zblk_hbm,
                  out_hbm, src_v, dst_v, rows_a, rows_b, acc_sh, sem_a, sem_b):
    cpt2 = src_v.shape[0]
    n_pad = acc_sh.shape[0]
    rows_per_tile = n_pad // NS
    cid = lax.axis_index("c")
    sid = lax.axis_index("s")
    wid = cid * NS + sid

    row0 = sid * rows_per_tile
    for r in range(rows_per_tile // CHUNK):
        pltpu.sync_copy(zblk_hbm, acc_sh.at[pl.ds(row0 + r * CHUNK, CHUNK)])
    plsc.subcore_barrier()

    bufs = (rows_a, rows_b)
    sems = (sem_a, sem_b)
    for half in range(2):
        chunk0 = pl.multiple_of(wid * cpt + half * cpt2, 8)

        @pl.when(wid < last_tile)
        def _stage_real():
            pltpu.sync_copy(ei_hbm.at[0, pl.ds(chunk0, cpt2)], src_v)
            pltpu.sync_copy(ei_hbm.at[1, pl.ds(chunk0, cpt2)], dst_v)

        @pl.when(wid >= last_tile)
        def _stage_tail():
            toff = pl.multiple_of(chunk0 - last_tile * cpt, 8)
            pltpu.sync_copy(tsrc_hbm.at[pl.ds(toff, cpt2)], src_v)
            pltpu.sync_copy(tdst_hbm.at[pl.ds(toff, cpt2)], dst_v)
        pltpu.async_copy(h_hbm.at[src_v.at[0]], rows_a, sem_a)
        pltpu.async_copy(h_hbm.at[src_v.at[1]], rows_b, sem_b)

        def pair_body(j2, carry):
            for b in range(2):
                j = 2 * j2 + b
                pltpu.make_async_copy(
                    h_hbm.at[src_v.at[j]], bufs[b], sems[b]).wait()
                pltpu.sync_copy(bufs[b], acc_sh.at[dst_v.at[j]], add=True)
                pltpu.async_copy(h_hbm.at[src_v.at[j + 2]], bufs[b], sems[b])
            return carry

        lax.fori_loop(0, cpt2 // 2 - 1, pair_body, 0)
        for b in range(2):
            j = cpt2 - 2 + b
            pltpu.make_async_copy(
                h_hbm.at[src_v.at[j]], bufs[b], sems[b]).wait()
            pltpu.sync_copy(bufs[b], acc_sh.at[dst_v.at[j]], add=True)
    plsc.subcore_barrier()

    pltpu.sync_copy(acc_sh.at[pl.ds(row0, rows_per_tile)],
                    out_hbm.at[cid, pl.ds(row0, rows_per_tile)])


def _segment_sum_sc(h, edge_index, n_pad):
    n, d = h.shape
    e = edge_index.shape[1]
    cpt = -(-e // (NW * CHUNK))
    cpt = -(-cpt // 4) * 4
    ept = cpt * CHUNK
    e_pad = ept * NW
    assert e % CHUNK == 0
    last_tile = e // ept
    ei3 = edge_index.reshape(2, e // CHUNK, CHUNK)
    if last_tile < NW:
        n_fill = e_pad - e
        fill = jnp.arange(n_fill, dtype=jnp.int32)
        tail_src = jnp.concatenate(
            [edge_index[0, last_tile * ept:], fill % CHUNK]).reshape(-1, CHUNK)
        tail_dst = jnp.concatenate(
            [edge_index[1, last_tile * ept:],
             n + fill % (n_pad - n)]).reshape(-1, CHUNK)
    else:
        tail_src = jnp.zeros((cpt, CHUNK), jnp.int32)
        tail_dst = jnp.full((cpt, CHUNK), n, jnp.int32)
    zblk = jnp.zeros((CHUNK, d), jnp.float32)

    mesh = plsc.VectorSubcoreMesh(core_axis_name="c", subcore_axis_name="s",
                                  num_cores=NC, num_subcores=NS)
    grid_kernel = functools.partial(
        pl.kernel,
        out_type=jax.ShapeDtypeStruct((NC, n_pad, d), jnp.float32),
        mesh=mesh,
        scratch_types=[
            pltpu.VMEM((cpt // 2, CHUNK), jnp.int32),
            pltpu.VMEM((cpt // 2, CHUNK), jnp.int32),
            pltpu.VMEM((CHUNK, d), jnp.float32),
            pltpu.VMEM((CHUNK, d), jnp.float32),
            pltpu.VMEM_SHARED((n_pad, d), jnp.float32),
            pltpu.SemaphoreType.DMA,
            pltpu.SemaphoreType.DMA,
        ],
    )
    body = functools.partial(_seg_sum_body, last_tile, cpt)
    return grid_kernel(body)(h, ei3, tail_src, tail_dst, zblk)


def _gin_tc_body(h_ref, parts_ref, w1_ref, b1_ref, w2_ref, b2_ref,
                 eps_ref, g_ref, bt_ref, out_ref):
    n = h_ref.shape[0]
    h = h_ref[...]
    neigh = parts_ref[0, :n, :] + parts_ref[1, :n, :]
    z = (1.0 + eps_ref[0, 0]) * h + neigh
    z = jnp.dot(z, w1_ref[...], preferred_element_type=jnp.float32)
    z = jnp.maximum(z + b1_ref[...], 0.0)
    z = jnp.dot(z, w2_ref[...], preferred_element_type=jnp.float32)
    z = z + b2_ref[...]
    mean = jnp.mean(z, axis=0, keepdims=True)
    var = jnp.mean((z - mean) ** 2, axis=0, keepdims=True)
    z = (z - mean) * lax.rsqrt(var + 1e-5) * g_ref[...] + bt_ref[...]
    out_ref[...] = h + jnp.maximum(z, 0.0)


def kernel(h, edge_index, W1, b1, W2, b2, eps, bn_gamma, bn_beta):
    n, d = h.shape
    n_pad = -(-(n + 1) // (NS * CHUNK)) * NS * CHUNK
    parts = _segment_sum_sc(h, edge_index, n_pad)
    return pl.pallas_call(
        _gin_tc_body,
        out_shape=jax.ShapeDtypeStruct((n, d), jnp.float32),
    )(h, parts,
      W1, b1.reshape(1, d), W2, b2.reshape(1, d),
      jnp.reshape(eps, (1, 1)), bn_gamma.reshape(1, d), bn_beta.reshape(1, d))
